# baseline (speedup 1.0000x reference)
import functools

import jax
import jax.numpy as jnp
from jax import lax
from jax.experimental import pallas as pl
from jax.experimental.pallas import tpu as pltpu
from jax.experimental.pallas import tpu_sc as plsc

E = 64
N_VECS = 8192
N_CODES = 8192
BZ = 1024
BK = 1024
NZ = N_VECS // BZ
NK = N_CODES // BK


def _vq_tc_body(z_ref, w_ref, wn_ref, idx_ref, loss_ref, mind_ref, argd_ref,
                acc_ref):
    i = pl.program_id(0)
    k = pl.program_id(1)

    zb = z_ref[...]
    zn = zb / jnp.maximum(
        jnp.sqrt(jnp.sum(zb * zb, axis=1, keepdims=True)), 1e-12)
    wb = w_ref[...]
    wn = wb / jnp.maximum(
        jnp.sqrt(jnp.sum(wb * wb, axis=1, keepdims=True)), 1e-12)
    wn_ref[...] = wn

    s = lax.dot_general(zn, wn, (((1,), (1,)), ((), ())),
                        preferred_element_type=jnp.float32)
    d = 2.0 - 2.0 * s
    dmin = jnp.min(d, axis=1, keepdims=True)
    col = lax.broadcasted_iota(jnp.int32, d.shape, 1) + k * BK
    amin = jnp.min(jnp.where(d == dmin, col, jnp.int32(2**30)),
                   axis=1, keepdims=True)

    @pl.when(k == 0)
    def _():
        mind_ref[...] = dmin
        argd_ref[...] = amin

    @pl.when(k > 0)
    def _():
        prev = mind_ref[...]
        better = dmin < prev
        argd_ref[...] = jnp.where(better, amin, argd_ref[...])
        mind_ref[...] = jnp.where(better, dmin, prev)

    @pl.when(k == NK - 1)
    def _():
        idx_ref[...] = argd_ref[...]
        part = jnp.sum(mind_ref[...])

        @pl.when(i == 0)
        def _():
            acc_ref[0, 0] = part

        @pl.when(i > 0)
        def _():
            acc_ref[0, 0] = acc_ref[0, 0] + part

        @pl.when(i == NZ - 1)
        def _():
            loss_ref[0, 0] = 1.25 * acc_ref[0, 0] / (N_VECS * E)


def _vq_tc(z2, w):
    return pl.pallas_call(
        _vq_tc_body,
        grid=(NZ, NK),
        in_specs=[
            pl.BlockSpec((BZ, E), lambda i, k: (i, 0)),
            pl.BlockSpec((BK, E), lambda i, k: (k, 0)),
        ],
        out_specs=[
            pl.BlockSpec((BK, E), lambda i, k: (k, 0)),
            pl.BlockSpec((BZ, 1), lambda i, k: (i, 0)),
            pl.BlockSpec((1, 1), lambda i, k: (0, 0),
                         memory_space=pltpu.SMEM),
        ],
        out_shape=[
            jax.ShapeDtypeStruct((N_CODES, E), jnp.float32),
            jax.ShapeDtypeStruct((N_VECS, 1), jnp.int32),
            jax.ShapeDtypeStruct((1, 1), jnp.float32),
        ],
        scratch_shapes=[
            pltpu.VMEM((BZ, 1), jnp.float32),
            pltpu.VMEM((BZ, 1), jnp.int32),
            pltpu.SMEM((1, 1), jnp.float32),
        ],
        compiler_params=pltpu.CompilerParams(
            dimension_semantics=("arbitrary", "arbitrary")),
    )(z2, w)


def _sc_gather(wn, idx_flat):
    info = plsc.get_sparse_core_info()
    nc, ns = info.num_cores, info.num_subcores
    nw = nc * ns
    per_w = N_VECS // nw
    ch = 128
    nch = per_w // ch
    mesh = plsc.VectorSubcoreMesh(core_axis_name="c", subcore_axis_name="s")

    @functools.partial(
        pl.kernel,
        mesh=mesh,
        out_type=jax.ShapeDtypeStruct((N_VECS, E), jnp.float32),
        scratch_types=[
            pltpu.VMEM((ch,), jnp.int32),
            pltpu.VMEM((ch, E), jnp.float32),
            pltpu.SemaphoreType.DMA,
        ],
        compiler_params=pltpu.CompilerParams(use_tc_tiling_on_sc=False),
    )
    def gather_k(table_hbm, idx_hbm, out_hbm, idx_v, rows_v, sem):
        wid = lax.axis_index("s") * nc + lax.axis_index("c")
        base = wid * per_w
        for c in range(nch):
            off = base + c * ch
            pltpu.sync_copy(idx_hbm.at[pl.ds(off, ch)], idx_v)
            pltpu.async_copy(table_hbm.at[idx_v], rows_v, sem).wait()
            pltpu.sync_copy(rows_v, out_hbm.at[pl.ds(off, ch)])

    return gather_k(wn, idx_flat)


def kernel(z, W):
    z2 = z.reshape(-1, E)
    wn, idx2, loss2 = _vq_tc(z2, W)
    zq = _sc_gather(wn, idx2.reshape(-1))
    return (zq.reshape(z.shape), loss2[0, 0], idx2.reshape(z.shape[:-1]))


# speedup vs baseline: 1.3635x; 1.3635x over previous
import functools

import jax
import jax.numpy as jnp
from jax import lax
from jax.experimental import pallas as pl
from jax.experimental.pallas import tpu as pltpu
from jax.experimental.pallas import tpu_sc as plsc

E = 64
KA = 128
N_VECS = 8192
N_CODES = 8192
BZ = 1024
BK = 2048
NZ = N_VECS // BZ
NK = N_CODES // BK
BIGF = 3.0e38


def _prep_body(z_ref, w_ref, lhs_ref, rhs_ref, wn_ref):
    zb = z_ref[...]
    zn = zb / jnp.maximum(
        jnp.sqrt(jnp.sum(zb * zb, axis=1, keepdims=True)), 1e-12)
    wb = w_ref[...]
    wn = wb / jnp.maximum(
        jnp.sqrt(jnp.sum(wb * wb, axis=1, keepdims=True)), 1e-12)
    wn_ref[...] = wn
    n = zb.shape[0]
    lhs_ref[...] = jnp.concatenate(
        [zn, jnp.ones((n, 1), jnp.float32),
         jnp.zeros((n, KA - E - 1), jnp.float32)], axis=1)
    rhs_ref[...] = jnp.concatenate(
        [-2.0 * wn, jnp.full((n, 1), 2.0, jnp.float32),
         jnp.zeros((n, KA - E - 1), jnp.float32)], axis=1)


def _prep(z2, w):
    return pl.pallas_call(
        _prep_body,
        grid=(NZ,),
        in_specs=[
            pl.BlockSpec((BZ, E), lambda r: (r, 0)),
            pl.BlockSpec((BZ, E), lambda r: (r, 0)),
        ],
        out_specs=[
            pl.BlockSpec((BZ, KA), lambda r: (r, 0)),
            pl.BlockSpec((BZ, KA), lambda r: (r, 0)),
            pl.BlockSpec((BZ, E), lambda r: (r, 0)),
        ],
        out_shape=[
            jax.ShapeDtypeStruct((N_VECS, KA), jnp.float32),
            jax.ShapeDtypeStruct((N_CODES, KA), jnp.float32),
            jax.ShapeDtypeStruct((N_CODES, E), jnp.float32),
        ],
    )(z2, w)


def _vq_tc_body(lhs_ref, rhs_ref, idx_ref, loss_ref, mind_ref, argf_ref,
                colf_ref, acc_ref):
    i = pl.program_id(0)
    k = pl.program_id(1)

    @pl.when((i == 0) & (k == 0))
    def _():
        colf_ref[...] = lax.broadcasted_iota(
            jnp.int32, (BZ, BK), 1).astype(jnp.float32)

    d = lax.dot_general(lhs_ref[...], rhs_ref[...], (((1,), (1,)), ((), ())),
                        preferred_element_type=jnp.float32)
    dmin = jnp.min(d, axis=1, keepdims=True)
    amin_local = jnp.min(jnp.where(d == dmin, colf_ref[...], BIGF),
                         axis=1, keepdims=True)
    aminf = amin_local + lax.convert_element_type(k * BK, jnp.float32)

    @pl.when(k == 0)
    def _():
        mind_ref[...] = dmin
        argf_ref[...] = aminf

    @pl.when(k > 0)
    def _():
        prev = mind_ref[...]
        better = dmin < prev
        argf_ref[...] = jnp.where(better, aminf, argf_ref[...])
        mind_ref[...] = jnp.where(better, dmin, prev)

    @pl.when(k == NK - 1)
    def _():
        idx_ref[...] = argf_ref[...].astype(jnp.int32)
        part = jnp.sum(mind_ref[...])

        @pl.when(i == 0)
        def _():
            acc_ref[0, 0] = part

        @pl.when(i > 0)
        def _():
            acc_ref[0, 0] = acc_ref[0, 0] + part

        @pl.when(i == NZ - 1)
        def _():
            loss_ref[0, 0] = 1.25 * acc_ref[0, 0] / (N_VECS * E)


def _vq_tc(lhsa, rhsa):
    return pl.pallas_call(
        _vq_tc_body,
        grid=(NZ, NK),
        in_specs=[
            pl.BlockSpec((BZ, KA), lambda i, k: (i, 0)),
            pl.BlockSpec((BK, KA), lambda i, k: (k, 0)),
        ],
        out_specs=[
            pl.BlockSpec((BZ, 1), lambda i, k: (i, 0)),
            pl.BlockSpec((1, 1), lambda i, k: (0, 0),
                         memory_space=pltpu.SMEM),
        ],
        out_shape=[
            jax.ShapeDtypeStruct((N_VECS, 1), jnp.int32),
            jax.ShapeDtypeStruct((1, 1), jnp.float32),
        ],
        scratch_shapes=[
            pltpu.VMEM((BZ, 1), jnp.float32),
            pltpu.VMEM((BZ, 1), jnp.float32),
            pltpu.VMEM((BZ, BK), jnp.float32),
            pltpu.SMEM((1, 1), jnp.float32),
        ],
        compiler_params=pltpu.CompilerParams(
            dimension_semantics=("arbitrary", "arbitrary"),
            vmem_limit_bytes=100 * 1024 * 1024),
    )(lhsa, rhsa)


def _sc_gather(wn, idx_flat):
    info = plsc.get_sparse_core_info()
    nc, ns = info.num_cores, info.num_subcores
    nw = nc * ns
    per_w = N_VECS // nw
    ch = 128
    nch = per_w // ch
    mesh = plsc.VectorSubcoreMesh(core_axis_name="c", subcore_axis_name="s")

    @functools.partial(
        pl.kernel,
        mesh=mesh,
        out_type=jax.ShapeDtypeStruct((N_VECS, E), jnp.float32),
        scratch_types=[
            pltpu.VMEM((ch,), jnp.int32),
            pltpu.VMEM((ch, E), jnp.float32),
            pltpu.SemaphoreType.DMA,
        ],
        compiler_params=pltpu.CompilerParams(use_tc_tiling_on_sc=False),
    )
    def gather_k(table_hbm, idx_hbm, out_hbm, idx_v, rows_v, sem):
        wid = lax.axis_index("s") * nc + lax.axis_index("c")
        base = wid * per_w
        for c in range(nch):
            off = base + c * ch
            pltpu.sync_copy(idx_hbm.at[pl.ds(off, ch)], idx_v)
            pltpu.async_copy(table_hbm.at[idx_v], rows_v, sem).wait()
            pltpu.sync_copy(rows_v, out_hbm.at[pl.ds(off, ch)])

    return gather_k(wn, idx_flat)


def kernel(z, W):
    z2 = z.reshape(-1, E)
    lhsa, rhsa, wn = _prep(z2, W)
    idx2, loss2 = _vq_tc(lhsa, rhsa)
    zq = _sc_gather(wn, idx2.reshape(-1))
    return (zq.reshape(z.shape), loss2[0, 0], idx2.reshape(z.shape[:-1]))


# speedup vs baseline: 1.7674x; 1.2963x over previous
import functools

import jax
import jax.numpy as jnp
from jax import lax
from jax.experimental import pallas as pl
from jax.experimental.pallas import tpu as pltpu
from jax.experimental.pallas import tpu_sc as plsc

E = 64
KA = 128
N_VECS = 8192
N_CODES = 8192
BZ = 2048
BK = 2048
NZ = N_VECS // BZ
NK = N_CODES // BK
BIGF = 3.0e38


def _prep_body(z_ref, w_ref, lhs_ref, rhs_ref, wn_ref):
    zb = z_ref[...]
    zn = zb / jnp.maximum(
        jnp.sqrt(jnp.sum(zb * zb, axis=1, keepdims=True)), 1e-12)
    wb = w_ref[...]
    wn = wb / jnp.maximum(
        jnp.sqrt(jnp.sum(wb * wb, axis=1, keepdims=True)), 1e-12)
    wn_ref[...] = wn
    n = zb.shape[0]
    lhs_ref[...] = jnp.concatenate(
        [zn, jnp.ones((n, 1), jnp.float32),
         jnp.zeros((n, KA - E - 1), jnp.float32)], axis=1)
    rhs_ref[...] = jnp.concatenate(
        [-2.0 * wn, jnp.full((n, 1), 2.0, jnp.float32),
         jnp.zeros((n, KA - E - 1), jnp.float32)], axis=1)


def _prep(z2, w):
    return pl.pallas_call(
        _prep_body,
        grid=(NZ,),
        in_specs=[
            pl.BlockSpec((BZ, E), lambda r: (r, 0)),
            pl.BlockSpec((BZ, E), lambda r: (r, 0)),
        ],
        out_specs=[
            pl.BlockSpec((BZ, KA), lambda r: (r, 0)),
            pl.BlockSpec((BZ, KA), lambda r: (r, 0)),
            pl.BlockSpec((BZ, E), lambda r: (r, 0)),
        ],
        out_shape=[
            jax.ShapeDtypeStruct((N_VECS, KA), jnp.float32),
            jax.ShapeDtypeStruct((N_CODES, KA), jnp.float32),
            jax.ShapeDtypeStruct((N_CODES, E), jnp.float32),
        ],
    )(z2, w)


def _vq_tc_body(lhs_ref, rhs_ref, idx_ref, loss_ref, mind_ref, argf_ref,
                acc_ref):
    i = pl.program_id(0)
    k = pl.program_id(1)

    d = lax.dot_general(lhs_ref[...], rhs_ref[...], (((1,), (1,)), ((), ())),
                        preferred_element_type=jnp.float32)
    ngrp = BK // 128
    rv = d[:, 0:128]
    rj = jnp.zeros((BZ, 128), jnp.float32)
    for j in range(1, ngrp):
        dj = d[:, j * 128:(j + 1) * 128]
        better = dj < rv
        rv = jnp.minimum(rv, dj)
        rj = jnp.where(better, jnp.full((BZ, 128), float(j), jnp.float32),
                       rj)
    dmin = jnp.min(rv, axis=1, keepdims=True)
    lanef = lax.broadcasted_iota(jnp.int32, (BZ, 128), 1).astype(jnp.float32)
    colg = rj * 128.0 + lanef
    amin_local = jnp.min(jnp.where(rv == dmin, colg, BIGF),
                         axis=1, keepdims=True)
    aminf = amin_local + lax.convert_element_type(k * BK, jnp.float32)

    @pl.when(k == 0)
    def _():
        mind_ref[...] = dmin
        argf_ref[...] = aminf

    @pl.when(k > 0)
    def _():
        prev = mind_ref[...]
        better = dmin < prev
        argf_ref[...] = jnp.where(better, aminf, argf_ref[...])
        mind_ref[...] = jnp.where(better, dmin, prev)

    @pl.when(k == NK - 1)
    def _():
        idx_ref[...] = argf_ref[...].astype(jnp.int32)
        part = jnp.sum(mind_ref[...])

        @pl.when(i == 0)
        def _():
            acc_ref[0, 0] = part

        @pl.when(i > 0)
        def _():
            acc_ref[0, 0] = acc_ref[0, 0] + part

        @pl.when(i == NZ - 1)
        def _():
            loss_ref[0, 0] = 1.25 * acc_ref[0, 0] / (N_VECS * E)


def _vq_tc(lhsa, rhsa):
    return pl.pallas_call(
        _vq_tc_body,
        grid=(NZ, NK),
        in_specs=[
            pl.BlockSpec((BZ, KA), lambda i, k: (i, 0)),
            pl.BlockSpec((BK, KA), lambda i, k: (k, 0)),
        ],
        out_specs=[
            pl.BlockSpec((BZ, 1), lambda i, k: (i, 0)),
            pl.BlockSpec((1, 1), lambda i, k: (0, 0),
                         memory_space=pltpu.SMEM),
        ],
        out_shape=[
            jax.ShapeDtypeStruct((N_VECS, 1), jnp.int32),
            jax.ShapeDtypeStruct((1, 1), jnp.float32),
        ],
        scratch_shapes=[
            pltpu.VMEM((BZ, 1), jnp.float32),
            pltpu.VMEM((BZ, 1), jnp.float32),
            pltpu.SMEM((1, 1), jnp.float32),
        ],
        compiler_params=pltpu.CompilerParams(
            dimension_semantics=("arbitrary", "arbitrary"),
            vmem_limit_bytes=100 * 1024 * 1024),
    )(lhsa, rhsa)


def _sc_gather(wn, idx_flat):
    info = plsc.get_sparse_core_info()
    nc, ns = info.num_cores, info.num_subcores
    nw = nc * ns
    per_w = N_VECS // nw
    ch = 128
    nch = per_w // ch
    mesh = plsc.VectorSubcoreMesh(core_axis_name="c", subcore_axis_name="s")

    @functools.partial(
        pl.kernel,
        mesh=mesh,
        out_type=jax.ShapeDtypeStruct((N_VECS, E), jnp.float32),
        scratch_types=[
            pltpu.VMEM((ch,), jnp.int32),
            pltpu.VMEM((ch, E), jnp.float32),
            pltpu.SemaphoreType.DMA,
        ],
        compiler_params=pltpu.CompilerParams(use_tc_tiling_on_sc=False),
    )
    def gather_k(table_hbm, idx_hbm, out_hbm, idx_v, rows_v, sem):
        wid = lax.axis_index("s") * nc + lax.axis_index("c")
        base = wid * per_w
        for c in range(nch):
            off = base + c * ch
            pltpu.sync_copy(idx_hbm.at[pl.ds(off, ch)], idx_v)
            pltpu.async_copy(table_hbm.at[idx_v], rows_v, sem).wait()
            pltpu.sync_copy(rows_v, out_hbm.at[pl.ds(off, ch)])

    return gather_k(wn, idx_flat)


def kernel(z, W):
    z2 = z.reshape(-1, E)
    lhsa, rhsa, wn = _prep(z2, W)
    idx2, loss2 = _vq_tc(lhsa, rhsa)
    zq = _sc_gather(wn, idx2.reshape(-1))
    return (zq.reshape(z.shape), loss2[0, 0], idx2.reshape(z.shape[:-1]))


# speedup vs baseline: 1.8778x; 1.0625x over previous
import functools

import jax
import jax.numpy as jnp
from jax import lax
from jax.experimental import pallas as pl
from jax.experimental.pallas import tpu as pltpu
from jax.experimental.pallas import tpu_sc as plsc

E = 64
KA = 128
N_VECS = 8192
N_CODES = 8192
BZ = 2048
BK = 2048
NZ = N_VECS // BZ
NK = N_CODES // BK
BIGF = 3.0e38


def _prep_body(z_ref, w_ref, lhs_ref, rhs_ref, wn_ref):
    zb = z_ref[...]
    zn = zb / jnp.maximum(
        jnp.sqrt(jnp.sum(zb * zb, axis=1, keepdims=True)), 1e-12)
    wb = w_ref[...]
    wn = wb / jnp.maximum(
        jnp.sqrt(jnp.sum(wb * wb, axis=1, keepdims=True)), 1e-12)
    wn_ref[...] = wn
    n = zb.shape[0]
    lhs_ref[...] = jnp.concatenate(
        [zn, jnp.ones((n, 1), jnp.float32),
         jnp.zeros((n, KA - E - 1), jnp.float32)], axis=1)
    rhs_ref[...] = jnp.concatenate(
        [-2.0 * wn, jnp.full((n, 1), 2.0, jnp.float32),
         jnp.zeros((n, KA - E - 1), jnp.float32)], axis=1)


def _prep(z2, w):
    return pl.pallas_call(
        _prep_body,
        grid=(NZ,),
        in_specs=[
            pl.BlockSpec((BZ, E), lambda r: (r, 0)),
            pl.BlockSpec((BZ, E), lambda r: (r, 0)),
        ],
        out_specs=[
            pl.BlockSpec((BZ, KA), lambda r: (r, 0)),
            pl.BlockSpec((BZ, KA), lambda r: (r, 0)),
            pl.BlockSpec((BZ, E), lambda r: (r, 0)),
        ],
        out_shape=[
            jax.ShapeDtypeStruct((N_VECS, KA), jnp.float32),
            jax.ShapeDtypeStruct((N_CODES, KA), jnp.float32),
            jax.ShapeDtypeStruct((N_CODES, E), jnp.float32),
        ],
    )(z2, w)


def _vq_tc_body(lhs_ref, rhs_ref, idx_ref, loss_ref, rv_ref, rj_ref,
                acc_ref):
    i = pl.program_id(0)
    k = pl.program_id(1)

    @pl.when(k == 0)
    def _():
        rv_ref[...] = jnp.full((BZ, 128), BIGF, jnp.float32)
        rj_ref[...] = jnp.zeros((BZ, 128), jnp.float32)

    d = lax.dot_general(lhs_ref[...], rhs_ref[...], (((1,), (1,)), ((), ())),
                        preferred_element_type=jnp.float32)
    ngrp = BK // 128
    basef = lax.convert_element_type(k * ngrp, jnp.float32)
    rv = rv_ref[...]
    rj = rj_ref[...]
    for j in range(ngrp):
        dj = d[:, j * 128:(j + 1) * 128]
        better = dj < rv
        rv = jnp.minimum(rv, dj)
        rj = jnp.where(better, basef + float(j), rj)
    rv_ref[...] = rv
    rj_ref[...] = rj

    @pl.when(k == NK - 1)
    def _():
        dmin = jnp.min(rv, axis=1, keepdims=True)
        lanef = lax.broadcasted_iota(
            jnp.int32, (BZ, 128), 1).astype(jnp.float32)
        colg = rj * 128.0 + lanef
        amin = jnp.min(jnp.where(rv == dmin, colg, BIGF),
                       axis=1, keepdims=True)
        idx_ref[...] = amin.astype(jnp.int32)
        part = jnp.sum(dmin)

        @pl.when(i == 0)
        def _():
            acc_ref[0, 0] = part

        @pl.when(i > 0)
        def _():
            acc_ref[0, 0] = acc_ref[0, 0] + part

        @pl.when(i == NZ - 1)
        def _():
            loss_ref[0, 0] = 1.25 * acc_ref[0, 0] / (N_VECS * E)


def _vq_tc(lhsa, rhsa):
    return pl.pallas_call(
        _vq_tc_body,
        grid=(NZ, NK),
        in_specs=[
            pl.BlockSpec((BZ, KA), lambda i, k: (i, 0)),
            pl.BlockSpec((BK, KA), lambda i, k: (k, 0)),
        ],
        out_specs=[
            pl.BlockSpec((BZ, 1), lambda i, k: (i, 0)),
            pl.BlockSpec((1, 1), lambda i, k: (0, 0),
                         memory_space=pltpu.SMEM),
        ],
        out_shape=[
            jax.ShapeDtypeStruct((N_VECS, 1), jnp.int32),
            jax.ShapeDtypeStruct((1, 1), jnp.float32),
        ],
        scratch_shapes=[
            pltpu.VMEM((BZ, 128), jnp.float32),
            pltpu.VMEM((BZ, 128), jnp.float32),
            pltpu.SMEM((1, 1), jnp.float32),
        ],
        compiler_params=pltpu.CompilerParams(
            dimension_semantics=("arbitrary", "arbitrary"),
            vmem_limit_bytes=100 * 1024 * 1024),
    )(lhsa, rhsa)


def _sc_gather(wn, idx_flat):
    info = plsc.get_sparse_core_info()
    nc, ns = info.num_cores, info.num_subcores
    nw = nc * ns
    per_w = N_VECS // nw
    ch = 128
    nch = per_w // ch
    mesh = plsc.VectorSubcoreMesh(core_axis_name="c", subcore_axis_name="s")

    @functools.partial(
        pl.kernel,
        mesh=mesh,
        out_type=jax.ShapeDtypeStruct((N_VECS, E), jnp.float32),
        scratch_types=[
            pltpu.VMEM((ch,), jnp.int32),
            pltpu.VMEM((ch, E), jnp.float32),
            pltpu.SemaphoreType.DMA,
        ],
        compiler_params=pltpu.CompilerParams(use_tc_tiling_on_sc=False),
    )
    def gather_k(table_hbm, idx_hbm, out_hbm, idx_v, rows_v, sem):
        wid = lax.axis_index("s") * nc + lax.axis_index("c")
        base = wid * per_w
        for c in range(nch):
            off = base + c * ch
            pltpu.sync_copy(idx_hbm.at[pl.ds(off, ch)], idx_v)
            pltpu.async_copy(table_hbm.at[idx_v], rows_v, sem).wait()
            pltpu.sync_copy(rows_v, out_hbm.at[pl.ds(off, ch)])

    return gather_k(wn, idx_flat)


def kernel(z, W):
    z2 = z.reshape(-1, E)
    lhsa, rhsa, wn = _prep(z2, W)
    idx2, loss2 = _vq_tc(lhsa, rhsa)
    zq = _sc_gather(wn, idx2.reshape(-1))
    return (zq.reshape(z.shape), loss2[0, 0], idx2.reshape(z.shape[:-1]))


# speedup vs baseline: 1.9886x; 1.0590x over previous
import functools

import jax
import jax.numpy as jnp
from jax import lax
from jax.experimental import pallas as pl
from jax.experimental.pallas import tpu as pltpu
from jax.experimental.pallas import tpu_sc as plsc

E = 64
KA = 128
N_VECS = 8192
N_CODES = 8192
BZ = 2048
BK = 2048
NZ = N_VECS // BZ
NK = N_CODES // BK
BIGF = 3.0e38


def _vq_tc_body(z_ref, w_ref, wn_ref, idx_ref, loss_ref, lhs_ref, rhs_ref,
                rv_ref, rj_ref, acc_ref):
    i = pl.program_id(0)
    k = pl.program_id(1)

    @pl.when(i == 0)
    def _():
        wb = w_ref[...]
        wnb = wb / jnp.maximum(
            jnp.sqrt(jnp.sum(wb * wb, axis=1, keepdims=True)), 1e-12)
        rhs_ref[pl.ds(k * BK, BK), :] = jnp.concatenate(
            [-2.0 * wnb, jnp.full((BK, 1), 2.0, jnp.float32),
             jnp.zeros((BK, KA - E - 1), jnp.float32)], axis=1)

    @pl.when(k == 0)
    def _():
        zb = z_ref[...]
        znb = zb / jnp.maximum(
            jnp.sqrt(jnp.sum(zb * zb, axis=1, keepdims=True)), 1e-12)
        lhs_ref[...] = jnp.concatenate(
            [znb, jnp.ones((BZ, 1), jnp.float32),
             jnp.zeros((BZ, KA - E - 1), jnp.float32)], axis=1)
        rv_ref[...] = jnp.full((BZ, 128), BIGF, jnp.float32)
        rj_ref[...] = jnp.zeros((BZ, 128), jnp.float32)

    wn_ref[...] = rhs_ref[pl.ds(k * BK, BK), :E] * -0.5

    d = lax.dot_general(lhs_ref[...], rhs_ref[pl.ds(k * BK, BK), :],
                        (((1,), (1,)), ((), ())),
                        preferred_element_type=jnp.float32)
    ngrp = BK // 128
    basef = lax.convert_element_type(k * ngrp, jnp.float32)
    rv = rv_ref[...]
    rj = rj_ref[...]
    for j in range(ngrp):
        dj = d[:, j * 128:(j + 1) * 128]
        better = dj < rv
        rv = jnp.minimum(rv, dj)
        rj = jnp.where(better, basef + float(j), rj)
    rv_ref[...] = rv
    rj_ref[...] = rj

    @pl.when(k == NK - 1)
    def _():
        dmin = jnp.min(rv, axis=1, keepdims=True)
        lanef = lax.broadcasted_iota(
            jnp.int32, (BZ, 128), 1).astype(jnp.float32)
        colg = rj * 128.0 + lanef
        amin = jnp.min(jnp.where(rv == dmin, colg, BIGF),
                       axis=1, keepdims=True)
        idx_ref[...] = amin.astype(jnp.int32)
        part = jnp.sum(dmin)

        @pl.when(i == 0)
        def _():
            acc_ref[0, 0] = part

        @pl.when(i > 0)
        def _():
            acc_ref[0, 0] = acc_ref[0, 0] + part

        @pl.when(i == NZ - 1)
        def _():
            loss_ref[0, 0] = 1.25 * acc_ref[0, 0] / (N_VECS * E)


def _vq_tc(z2, w):
    return pl.pallas_call(
        _vq_tc_body,
        grid=(NZ, NK),
        in_specs=[
            pl.BlockSpec((BZ, E), lambda i, k: (i, 0)),
            pl.BlockSpec((BK, E), lambda i, k: (k, 0)),
        ],
        out_specs=[
            pl.BlockSpec((BK, E), lambda i, k: (k, 0)),
            pl.BlockSpec((BZ, 1), lambda i, k: (i, 0)),
            pl.BlockSpec((1, 1), lambda i, k: (0, 0),
                         memory_space=pltpu.SMEM),
        ],
        out_shape=[
            jax.ShapeDtypeStruct((N_CODES, E), jnp.float32),
            jax.ShapeDtypeStruct((N_VECS, 1), jnp.int32),
            jax.ShapeDtypeStruct((1, 1), jnp.float32),
        ],
        scratch_shapes=[
            pltpu.VMEM((BZ, KA), jnp.float32),
            pltpu.VMEM((N_CODES, KA), jnp.float32),
            pltpu.VMEM((BZ, 128), jnp.float32),
            pltpu.VMEM((BZ, 128), jnp.float32),
            pltpu.SMEM((1, 1), jnp.float32),
        ],
        compiler_params=pltpu.CompilerParams(
            dimension_semantics=("arbitrary", "arbitrary"),
            vmem_limit_bytes=100 * 1024 * 1024),
    )(z2, w)


def _sc_gather(wn, idx_flat):
    info = plsc.get_sparse_core_info()
    nc, ns = info.num_cores, info.num_subcores
    nw = nc * ns
    per_w = N_VECS // nw
    ch = 128
    nch = per_w // ch
    mesh = plsc.VectorSubcoreMesh(core_axis_name="c", subcore_axis_name="s")

    @functools.partial(
        pl.kernel,
        mesh=mesh,
        out_type=jax.ShapeDtypeStruct((N_VECS, E), jnp.float32),
        scratch_types=[
            pltpu.VMEM((ch,), jnp.int32),
            pltpu.VMEM((ch, E), jnp.float32),
            pltpu.SemaphoreType.DMA,
        ],
        compiler_params=pltpu.CompilerParams(use_tc_tiling_on_sc=False),
    )
    def gather_k(table_hbm, idx_hbm, out_hbm, idx_v, rows_v, sem):
        wid = lax.axis_index("s") * nc + lax.axis_index("c")
        base = wid * per_w
        for c in range(nch):
            off = base + c * ch
            pltpu.sync_copy(idx_hbm.at[pl.ds(off, ch)], idx_v)
            pltpu.async_copy(table_hbm.at[idx_v], rows_v, sem).wait()
            pltpu.sync_copy(rows_v, out_hbm.at[pl.ds(off, ch)])

    return gather_k(wn, idx_flat)


def kernel(z, W):
    z2 = z.reshape(-1, E)
    wn, idx2, loss2 = _vq_tc(z2, W)
    zq = _sc_gather(wn, idx2.reshape(-1))
    return (zq.reshape(z.shape), loss2[0, 0], idx2.reshape(z.shape[:-1]))


# speedup vs baseline: 2.0377x; 1.0247x over previous
import functools

import jax
import jax.numpy as jnp
from jax import lax
from jax.experimental import pallas as pl
from jax.experimental.pallas import tpu as pltpu
from jax.experimental.pallas import tpu_sc as plsc

E = 64
KA = 128
N_VECS = 8192
N_CODES = 8192
BZ = 2048
BK = 2048
NZ = N_VECS // BZ
NK = N_CODES // BK
BIGF = 3.0e38


def _vq_tc_body(z_ref, w_ref, wn_ref, idx_ref, loss_ref, lhs_ref, rhs_ref,
                rv_ref, rj_ref, acc_ref):
    i = pl.program_id(0)
    k = pl.program_id(1)

    @pl.when(i == 0)
    def _():
        wb = w_ref[...]
        wnb = wb / jnp.maximum(
            jnp.sqrt(jnp.sum(wb * wb, axis=1, keepdims=True)), 1e-12)
        rhs_ref[pl.ds(k * BK, BK), :] = jnp.concatenate(
            [-2.0 * wnb, jnp.full((BK, 1), 2.0, jnp.float32),
             jnp.zeros((BK, KA - E - 1), jnp.float32)], axis=1)

    @pl.when(k == 0)
    def _():
        zb = z_ref[...]
        znb = zb / jnp.maximum(
            jnp.sqrt(jnp.sum(zb * zb, axis=1, keepdims=True)), 1e-12)
        lhs_ref[...] = jnp.concatenate(
            [znb, jnp.ones((BZ, 1), jnp.float32),
             jnp.zeros((BZ, KA - E - 1), jnp.float32)], axis=1)
        rv_ref[...] = jnp.full((BZ, 128), BIGF, jnp.float32)
        rj_ref[...] = jnp.zeros((BZ, 128), jnp.float32)

    wn_ref[...] = rhs_ref[pl.ds(k * BK, BK), :E] * -0.5

    d = lax.dot_general(lhs_ref[...], rhs_ref[pl.ds(k * BK, BK), :],
                        (((1,), (1,)), ((), ())),
                        preferred_element_type=jnp.float32)
    ngrp = BK // 128
    basef = lax.convert_element_type(k * ngrp, jnp.float32)
    rv = rv_ref[...]
    rj = rj_ref[...]
    for j in range(ngrp):
        dj = d[:, j * 128:(j + 1) * 128]
        better = dj < rv
        rv = jnp.minimum(rv, dj)
        rj = jnp.where(better, basef + float(j), rj)
    rv_ref[...] = rv
    rj_ref[...] = rj

    @pl.when(k == NK - 1)
    def _():
        dmin = jnp.min(rv, axis=1, keepdims=True)
        lanef = lax.broadcasted_iota(
            jnp.int32, (BZ, 128), 1).astype(jnp.float32)
        colg = rj * 128.0 + lanef
        amin = jnp.min(jnp.where(rv == dmin, colg, BIGF),
                       axis=1, keepdims=True)
        idx_ref[...] = amin.astype(jnp.int32).reshape(BZ // 128, 128)
        part = jnp.sum(dmin)

        @pl.when(i == 0)
        def _():
            acc_ref[0, 0] = part

        @pl.when(i > 0)
        def _():
            acc_ref[0, 0] = acc_ref[0, 0] + part

        @pl.when(i == NZ - 1)
        def _():
            loss_ref[0, 0] = 1.25 * acc_ref[0, 0] / (N_VECS * E)


def _vq_tc(z2, w):
    return pl.pallas_call(
        _vq_tc_body,
        grid=(NZ, NK),
        in_specs=[
            pl.BlockSpec((BZ, E), lambda i, k: (i, 0)),
            pl.BlockSpec((BK, E), lambda i, k: (k, 0)),
        ],
        out_specs=[
            pl.BlockSpec((BK, E), lambda i, k: (k, 0)),
            pl.BlockSpec((BZ // 128, 128), lambda i, k: (i, 0)),
            pl.BlockSpec((1, 1), lambda i, k: (0, 0),
                         memory_space=pltpu.SMEM),
        ],
        out_shape=[
            jax.ShapeDtypeStruct((N_CODES, E), jnp.float32),
            jax.ShapeDtypeStruct((N_VECS // 128, 128), jnp.int32),
            jax.ShapeDtypeStruct((1, 1), jnp.float32),
        ],
        scratch_shapes=[
            pltpu.VMEM((BZ, KA), jnp.float32),
            pltpu.VMEM((N_CODES, KA), jnp.float32),
            pltpu.VMEM((BZ, 128), jnp.float32),
            pltpu.VMEM((BZ, 128), jnp.float32),
            pltpu.SMEM((1, 1), jnp.float32),
        ],
        compiler_params=pltpu.CompilerParams(
            dimension_semantics=("arbitrary", "arbitrary"),
            vmem_limit_bytes=100 * 1024 * 1024),
    )(z2, w)


def _sc_gather(wn, idx_flat):
    info = plsc.get_sparse_core_info()
    nc, ns = info.num_cores, info.num_subcores
    nw = nc * ns
    per_w = N_VECS // nw
    ch = 128
    nch = per_w // ch
    mesh = plsc.VectorSubcoreMesh(core_axis_name="c", subcore_axis_name="s")

    @functools.partial(
        pl.kernel,
        mesh=mesh,
        out_type=jax.ShapeDtypeStruct((N_VECS, E), jnp.float32),
        scratch_types=[
            pltpu.VMEM((ch,), jnp.int32),
            pltpu.VMEM((ch, E), jnp.float32),
            pltpu.SemaphoreType.DMA,
        ],
        compiler_params=pltpu.CompilerParams(use_tc_tiling_on_sc=False),
    )
    def gather_k(table_hbm, idx_hbm, out_hbm, idx_v, rows_v, sem):
        wid = lax.axis_index("s") * nc + lax.axis_index("c")
        base = wid * per_w
        for c in range(nch):
            off = base + c * ch
            pltpu.sync_copy(idx_hbm.at[pl.ds(off, ch)], idx_v)
            pltpu.async_copy(table_hbm.at[idx_v], rows_v, sem).wait()
            pltpu.sync_copy(rows_v, out_hbm.at[pl.ds(off, ch)])

    return gather_k(wn, idx_flat)


def kernel(z, W):
    z2 = z.reshape(-1, E)
    wn, idx2, loss2 = _vq_tc(z2, W)
    zq = _sc_gather(wn, idx2.reshape(-1))
    return (zq.reshape(z.shape), loss2[0, 0],
            idx2.reshape(z.shape[:-1]))


# speedup vs baseline: 2.1336x; 1.0471x over previous
import functools

import jax
import jax.numpy as jnp
from jax import lax
from jax.experimental import pallas as pl
from jax.experimental.pallas import tpu as pltpu
from jax.experimental.pallas import tpu_sc as plsc

E = 64
KA = 128
N_VECS = 8192
N_CODES = 8192
BZ = 2048
BK = 2048
NZ = N_VECS // BZ
NK = N_CODES // BK
BIGF = 3.0e38


def _vq_tc_body(z_ref, w_ref, wn_ref, idx_ref, loss_ref, lhs_ref, rhs_ref,
                rv_ref, rj_ref, acc_ref):
    i = pl.program_id(0)
    k = pl.program_id(1)

    @pl.when(i == 0)
    def _():
        wb = w_ref[...]
        wnb = wb / jnp.maximum(
            jnp.sqrt(jnp.sum(wb * wb, axis=1, keepdims=True)), 1e-12)
        rhs_ref[pl.ds(k * BK, BK), :] = jnp.concatenate(
            [-2.0 * wnb, jnp.full((BK, 1), 2.0, jnp.float32),
             jnp.zeros((BK, KA - E - 1), jnp.float32)], axis=1)

    @pl.when(k == 0)
    def _():
        zb = z_ref[...]
        znb = zb / jnp.maximum(
            jnp.sqrt(jnp.sum(zb * zb, axis=1, keepdims=True)), 1e-12)
        lhs_ref[...] = jnp.concatenate(
            [znb, jnp.ones((BZ, 1), jnp.float32),
             jnp.zeros((BZ, KA - E - 1), jnp.float32)], axis=1)
        rv_ref[...] = jnp.full((BZ, 128), BIGF, jnp.float32)
        rj_ref[...] = jnp.zeros((BZ, 128), jnp.float32)

    step = i * NK + k
    srows = N_CODES // (NZ * NK)
    wn_ref[...] = rhs_ref[pl.ds(step * srows, srows), :] * -0.5

    d = lax.dot_general(lhs_ref[...], rhs_ref[pl.ds(k * BK, BK), :],
                        (((1,), (1,)), ((), ())),
                        preferred_element_type=jnp.float32)
    ngrp = BK // 128
    basef = lax.convert_element_type(k * ngrp, jnp.float32)
    rv = rv_ref[...]
    rj = rj_ref[...]
    for j in range(ngrp):
        dj = d[:, j * 128:(j + 1) * 128]
        better = dj < rv
        rv = jnp.minimum(rv, dj)
        rj = jnp.where(better, basef + float(j), rj)
    rv_ref[...] = rv
    rj_ref[...] = rj

    @pl.when(k == NK - 1)
    def _():
        dmin = jnp.min(rv, axis=1, keepdims=True)
        lanef = lax.broadcasted_iota(
            jnp.int32, (BZ, 128), 1).astype(jnp.float32)
        colg = rj * 128.0 + lanef
        amin = jnp.min(jnp.where(rv == dmin, colg, BIGF),
                       axis=1, keepdims=True)
        idx_ref[...] = amin.astype(jnp.int32).reshape(BZ // 128, 128)
        part = jnp.sum(dmin)

        @pl.when(i == 0)
        def _():
            acc_ref[0, 0] = part

        @pl.when(i > 0)
        def _():
            acc_ref[0, 0] = acc_ref[0, 0] + part

        @pl.when(i == NZ - 1)
        def _():
            loss_ref[0, 0] = 1.25 * acc_ref[0, 0] / (N_VECS * E)


def _vq_tc(z2, w):
    return pl.pallas_call(
        _vq_tc_body,
        grid=(NZ, NK),
        in_specs=[
            pl.BlockSpec((BZ, E), lambda i, k: (i, 0)),
            pl.BlockSpec((BK, E), lambda i, k: (k, 0)),
        ],
        out_specs=[
            pl.BlockSpec((N_CODES // (NZ * NK), KA),
                         lambda i, k: (i * NK + k, 0)),
            pl.BlockSpec((BZ // 128, 128), lambda i, k: (i, 0)),
            pl.BlockSpec((1, 1), lambda i, k: (0, 0),
                         memory_space=pltpu.SMEM),
        ],
        out_shape=[
            jax.ShapeDtypeStruct((N_CODES, KA), jnp.float32),
            jax.ShapeDtypeStruct((N_VECS // 128, 128), jnp.int32),
            jax.ShapeDtypeStruct((1, 1), jnp.float32),
        ],
        scratch_shapes=[
            pltpu.VMEM((BZ, KA), jnp.float32),
            pltpu.VMEM((N_CODES, KA), jnp.float32),
            pltpu.VMEM((BZ, 128), jnp.float32),
            pltpu.VMEM((BZ, 128), jnp.float32),
            pltpu.SMEM((1, 1), jnp.float32),
        ],
        compiler_params=pltpu.CompilerParams(
            dimension_semantics=("arbitrary", "arbitrary"),
            vmem_limit_bytes=100 * 1024 * 1024),
    )(z2, w)


def _sc_gather(wn, idx_flat):
    info = plsc.get_sparse_core_info()
    nc, ns = info.num_cores, info.num_subcores
    nw = nc * ns
    per_w = N_VECS // nw
    ch = 128
    nch = per_w // ch
    mesh = plsc.VectorSubcoreMesh(core_axis_name="c", subcore_axis_name="s")

    @functools.partial(
        pl.kernel,
        mesh=mesh,
        out_type=jax.ShapeDtypeStruct((N_VECS, E), jnp.float32),
        scratch_types=[
            pltpu.VMEM((ch,), jnp.int32),
            pltpu.VMEM((ch, KA), jnp.float32),
            pltpu.SemaphoreType.DMA,
        ],
        compiler_params=pltpu.CompilerParams(use_tc_tiling_on_sc=False),
    )
    def gather_k(table_hbm, idx_hbm, out_hbm, idx_v, rows_v, sem):
        wid = lax.axis_index("s") * nc + lax.axis_index("c")
        base = wid * per_w
        for c in range(nch):
            off = base + c * ch
            pltpu.sync_copy(idx_hbm.at[pl.ds(off, ch)], idx_v)
            pltpu.async_copy(table_hbm.at[idx_v], rows_v, sem).wait()
            pltpu.sync_copy(rows_v.at[:, pl.ds(0, E)],
                            out_hbm.at[pl.ds(off, ch)])

    return gather_k(wn, idx_flat)


def kernel(z, W):
    z2 = z.reshape(-1, E)
    wn, idx2, loss2 = _vq_tc(z2, W)
    zq = _sc_gather(wn, idx2.reshape(-1))
    return (zq.reshape(z.shape), loss2[0, 0],
            idx2.reshape(z.shape[:-1]))


# speedup vs baseline: 2.1916x; 1.0272x over previous
import functools

import jax
import jax.numpy as jnp
from jax import lax
from jax.experimental import pallas as pl
from jax.experimental.pallas import tpu as pltpu
from jax.experimental.pallas import tpu_sc as plsc

E = 64
KA = 128
N_VECS = 8192
N_CODES = 8192
BZ = 2048
BK = 4096
NZ = N_VECS // BZ
NK = N_CODES // BK
BIGF = 3.0e38


def _vq_tc_body(z_ref, w_ref, wn_ref, idx_ref, loss_ref, lhs_ref, rhs_ref,
                rv_ref, rj_ref, acc_ref):
    i = pl.program_id(0)
    k = pl.program_id(1)

    @pl.when(i == 0)
    def _():
        wb = w_ref[...]
        wnb = wb / jnp.maximum(
            jnp.sqrt(jnp.sum(wb * wb, axis=1, keepdims=True)), 1e-12)
        rhs_ref[pl.ds(k * BK, BK), :] = jnp.concatenate(
            [-2.0 * wnb, jnp.full((BK, 1), 2.0, jnp.float32),
             jnp.zeros((BK, KA - E - 1), jnp.float32)], axis=1)

    @pl.when(k == 0)
    def _():
        zb = z_ref[...]
        znb = zb / jnp.maximum(
            jnp.sqrt(jnp.sum(zb * zb, axis=1, keepdims=True)), 1e-12)
        lhs_ref[...] = jnp.concatenate(
            [znb, jnp.ones((BZ, 1), jnp.float32),
             jnp.zeros((BZ, KA - E - 1), jnp.float32)], axis=1)
        rv_ref[...] = jnp.full((BZ, 128), BIGF, jnp.float32)
        rj_ref[...] = jnp.zeros((BZ, 128), jnp.float32)

    step = i * NK + k
    srows = N_CODES // (NZ * NK)
    wn_ref[...] = rhs_ref[pl.ds(step * srows, srows), :] * -0.5

    d = lax.dot_general(lhs_ref[...], rhs_ref[pl.ds(k * BK, BK), :],
                        (((1,), (1,)), ((), ())),
                        preferred_element_type=jnp.float32)
    ngrp = BK // 128
    basef = lax.convert_element_type(k * ngrp, jnp.float32)
    rv = rv_ref[...]
    rj = rj_ref[...]
    for j in range(ngrp):
        dj = d[:, j * 128:(j + 1) * 128]
        better = dj < rv
        rv = jnp.minimum(rv, dj)
        rj = jnp.where(better, basef + float(j), rj)
    rv_ref[...] = rv
    rj_ref[...] = rj

    @pl.when(k == NK - 1)
    def _():
        dmin = jnp.min(rv, axis=1, keepdims=True)
        lanef = lax.broadcasted_iota(
            jnp.int32, (BZ, 128), 1).astype(jnp.float32)
        colg = rj * 128.0 + lanef
        amin = jnp.min(jnp.where(rv == dmin, colg, BIGF),
                       axis=1, keepdims=True)
        idx_ref[...] = amin.astype(jnp.int32).reshape(BZ // 128, 128)
        part = jnp.sum(dmin)

        @pl.when(i == 0)
        def _():
            acc_ref[0, 0] = part

        @pl.when(i > 0)
        def _():
            acc_ref[0, 0] = acc_ref[0, 0] + part

        @pl.when(i == NZ - 1)
        def _():
            loss_ref[0, 0] = 1.25 * acc_ref[0, 0] / (N_VECS * E)


def _vq_tc(z2, w):
    return pl.pallas_call(
        _vq_tc_body,
        grid=(NZ, NK),
        in_specs=[
            pl.BlockSpec((BZ, E), lambda i, k: (i, 0)),
            pl.BlockSpec((BK, E), lambda i, k: (k, 0)),
        ],
        out_specs=[
            pl.BlockSpec((N_CODES // (NZ * NK), KA),
                         lambda i, k: (i * NK + k, 0)),
            pl.BlockSpec((BZ // 128, 128), lambda i, k: (i, 0)),
            pl.BlockSpec((1, 1), lambda i, k: (0, 0),
                         memory_space=pltpu.SMEM),
        ],
        out_shape=[
            jax.ShapeDtypeStruct((N_CODES, KA), jnp.float32),
            jax.ShapeDtypeStruct((N_VECS // 128, 128), jnp.int32),
            jax.ShapeDtypeStruct((1, 1), jnp.float32),
        ],
        scratch_shapes=[
            pltpu.VMEM((BZ, KA), jnp.float32),
            pltpu.VMEM((N_CODES, KA), jnp.float32),
            pltpu.VMEM((BZ, 128), jnp.float32),
            pltpu.VMEM((BZ, 128), jnp.float32),
            pltpu.SMEM((1, 1), jnp.float32),
        ],
        compiler_params=pltpu.CompilerParams(
            dimension_semantics=("arbitrary", "arbitrary"),
            vmem_limit_bytes=100 * 1024 * 1024),
    )(z2, w)


def _sc_gather(wn, idx_flat):
    info = plsc.get_sparse_core_info()
    nc, ns = info.num_cores, info.num_subcores
    nw = nc * ns
    per_w = N_VECS // nw
    ch = 128
    nch = per_w // ch
    mesh = plsc.VectorSubcoreMesh(core_axis_name="c", subcore_axis_name="s")

    @functools.partial(
        pl.kernel,
        mesh=mesh,
        out_type=jax.ShapeDtypeStruct((N_VECS, E), jnp.float32),
        scratch_types=[
            pltpu.VMEM((ch,), jnp.int32),
            pltpu.VMEM((ch, KA), jnp.float32),
            pltpu.SemaphoreType.DMA,
        ],
        compiler_params=pltpu.CompilerParams(use_tc_tiling_on_sc=False),
    )
    def gather_k(table_hbm, idx_hbm, out_hbm, idx_v, rows_v, sem):
        wid = lax.axis_index("s") * nc + lax.axis_index("c")
        base = wid * per_w
        for c in range(nch):
            off = base + c * ch
            pltpu.sync_copy(idx_hbm.at[pl.ds(off, ch)], idx_v)
            pltpu.async_copy(table_hbm.at[idx_v], rows_v, sem).wait()
            pltpu.sync_copy(rows_v.at[:, pl.ds(0, E)],
                            out_hbm.at[pl.ds(off, ch)])

    return gather_k(wn, idx_flat)


def kernel(z, W):
    z2 = z.reshape(-1, E)
    wn, idx2, loss2 = _vq_tc(z2, W)
    zq = _sc_gather(wn, idx2.reshape(-1))
    return (zq.reshape(z.shape), loss2[0, 0],
            idx2.reshape(z.shape[:-1]))
